# baseline (speedup 1.0000x reference)
import functools

import jax
import jax.numpy as jnp
from jax import lax
from jax.experimental import pallas as pl
from jax.experimental.pallas import tpu as pltpu
from jax.experimental.pallas import tpu_sc as plsc

NC = 2
NS = 16
NW = NC * NS

IDX_PER_DMA = 128
DMAS_PER_BLOCK = 8
BLOCK = IDX_PER_DMA * DMAS_PER_BLOCK


def _gather_body(n_blocks, table_hbm, idx_hbm, out_hbm,
                 idx_v, rows0, rows1, sem0, sem1):
    wid = lax.axis_index("s") * NC + lax.axis_index("c")
    rows_per_w = n_blocks * BLOCK
    base = wid * rows_per_w

    pltpu.sync_copy(idx_hbm.at[wid], idx_v)

    bufs = (rows0, rows1)
    sems = (sem0, sem1)

    def fire(g, buf, sem):
        handles = []
        for j in range(DMAS_PER_BLOCK):
            chunk = g * DMAS_PER_BLOCK + j
            h = pltpu.async_copy(
                table_hbm.at[idx_v.at[chunk]],
                buf.at[pl.ds(j * IDX_PER_DMA, IDX_PER_DMA)],
                sem,
            )
            handles.append(h)
        return handles

    pending = fire(0, bufs[0], sems[0])
    for g in range(n_blocks):
        cur = pending
        if g + 1 < n_blocks:
            pending = fire(g + 1, bufs[(g + 1) % 2], sems[(g + 1) % 2])
        for h in cur:
            h.wait()
        pltpu.sync_copy(bufs[g % 2], out_hbm.at[pl.ds(base + g * BLOCK, BLOCK)])


def kernel(indices, weight):
    B, H = indices.shape
    V, D = weight.shape
    total = B * H
    assert total % (NW * BLOCK) == 0
    rows_per_w = total // NW
    n_blocks = rows_per_w // BLOCK
    n_chunks = rows_per_w // IDX_PER_DMA

    idx3 = indices.reshape(NW, n_chunks, IDX_PER_DMA)
    dw = D // 2
    w32 = lax.bitcast_convert_type(weight.reshape(V, dw, 2), jnp.int32)

    mesh = plsc.VectorSubcoreMesh(core_axis_name="c", subcore_axis_name="s")
    body = functools.partial(_gather_body, n_blocks)
    out = pl.kernel(
        body,
        out_type=jax.ShapeDtypeStruct((total, dw), jnp.int32),
        mesh=mesh,
        scratch_types=[
            pltpu.VMEM((n_chunks, IDX_PER_DMA), jnp.int32),
            pltpu.VMEM((BLOCK, dw), jnp.int32),
            pltpu.VMEM((BLOCK, dw), jnp.int32),
            pltpu.SemaphoreType.DMA,
            pltpu.SemaphoreType.DMA,
        ],
        compiler_params=pltpu.CompilerParams(use_tc_tiling_on_sc=False),
    )(w32, idx3)
    return lax.bitcast_convert_type(out, weight.dtype).reshape(B, H, D)


# speedup vs baseline: 1.6303x; 1.6303x over previous
import functools

import jax
import jax.numpy as jnp
from jax import lax
from jax.experimental import pallas as pl
from jax.experimental.pallas import tpu as pltpu
from jax.experimental.pallas import tpu_sc as plsc

NC = 2
NS = 16
NW = NC * NS
LANE = 128


def _body(V, D, B, H, wT, idxT, wtail, out, ltab,
          a_bf, a_tl, lb_v, idx_v, g_v, o_bf, sem_a, sem_l, sem_i, sem_g,
          sem_o, bsem):
    dw = D // 2
    wid = lax.axis_index("s") * NC + lax.axis_index("c")

    iotas = [lax.iota(jnp.int32, 16) + 16 * g for g in range(8)]
    cols = [jnp.full((16,), k, jnp.int32) for k in range(dw)]

    nblk = V // LANE
    tail = V - nblk * LANE
    per, rem = nblk // NW, nblk % NW
    start = wid * per + jnp.minimum(wid, rem)
    count = per + (wid < rem).astype(jnp.int32)

    a32 = a_bf.bitcast(jnp.int32)

    def phase_a(i, _):
        vc = start + i
        pltpu.async_copy(wT.at[:, pl.ds(vc * LANE, LANE)], a_bf, sem_a).wait()
        for k in range(dw):
            for g in range(8):
                plsc.store_scatter(lb_v, [iotas[g], cols[k]],
                                   a32[k, pl.ds(16 * g, 16)])
        pltpu.async_copy(lb_v, ltab.at[pl.ds(vc * LANE, LANE)], sem_l).wait()
        return 0

    lax.fori_loop(0, count, phase_a, 0)

    @pl.when(wid == NW - 1)
    def _tail():
        pltpu.async_copy(wtail, a_tl, sem_a).wait()
        t32 = a_tl.bitcast(jnp.int32)
        for k in range(dw):
            for g in range(tail // 16):
                plsc.store_scatter(lb_v, [iotas[g], cols[k]],
                                   t32[k, pl.ds(16 * g, 16)])
        pltpu.async_copy(lb_v.at[pl.ds(0, tail)],
                         ltab.at[pl.ds(nblk * LANE, tail)], sem_l).wait()

    for tc in range(NC):
        for ts in range(NS):
            pl.semaphore_signal(bsem, 1, device_id={"c": tc, "s": ts})
    pl.semaphore_wait(bsem, NW)

    ntb = B // LANE
    blocks_per_w = (H * ntb) // NW
    o32 = o_bf.bitcast(jnp.int32)

    def phase_b(j, _):
        fb = wid * blocks_per_w + j
        h = fb // ntb
        tb = fb % ntb
        pltpu.async_copy(idxT.at[h, pl.ds(tb * LANE, LANE)], idx_v,
                         sem_i).wait()
        pltpu.async_copy(ltab.at[idx_v], g_v, sem_g).wait()
        for k in range(dw):
            for g in range(8):
                o32[k, pl.ds(16 * g, 16)] = plsc.load_gather(
                    g_v, [iotas[g], cols[k]])
        pltpu.async_copy(o_bf, out.at[h, :, pl.ds(tb * LANE, LANE)],
                         sem_o).wait()
        return 0

    lax.fori_loop(0, blocks_per_w, phase_b, 0)


def kernel(indices, weight):
    B, H = indices.shape
    V, D = weight.shape
    dw = D // 2
    assert B % LANE == 0 and (H * (B // LANE)) % NW == 0

    nblk = V // LANE
    tail = V - nblk * LANE
    wT = weight.T
    wtail = lax.slice(wT, (0, nblk * LANE), (D, V))

    mesh = plsc.VectorSubcoreMesh(core_axis_name="c", subcore_axis_name="s")
    body = functools.partial(_body, V, D, B, H)
    out = pl.kernel(
        body,
        out_type=jax.ShapeDtypeStruct((H, D, B), weight.dtype),
        mesh=mesh,
        scratch_types=[
            pltpu.HBM((V, dw), jnp.int32),
            pltpu.VMEM((D, LANE), weight.dtype),
            pltpu.VMEM((D, tail), weight.dtype),
            pltpu.VMEM((LANE, dw), jnp.int32),
            pltpu.VMEM((LANE,), jnp.int32),
            pltpu.VMEM((LANE, dw), jnp.int32),
            pltpu.VMEM((D, LANE), weight.dtype),
            pltpu.SemaphoreType.DMA,
            pltpu.SemaphoreType.DMA,
            pltpu.SemaphoreType.DMA,
            pltpu.SemaphoreType.DMA,
            pltpu.SemaphoreType.DMA,
            pltpu.SemaphoreType.REGULAR,
        ],
        compiler_params=pltpu.CompilerParams(use_tc_tiling_on_sc=True,
                                             needs_layout_passes=False),
    )(wT, indices.T, wtail)
    return out.transpose(2, 0, 1)


# speedup vs baseline: 2.4707x; 1.5155x over previous
import functools

import jax
import jax.numpy as jnp
from jax import lax
from jax.experimental import pallas as pl
from jax.experimental.pallas import tpu as pltpu
from jax.experimental.pallas import tpu_sc as plsc

NC = 2
NS = 16
NW = NC * NS
LANE = 128
BLK_A = 128
BLK_B = 128


def _body(V, D, B, H, wT, idxT, wtail, out, ltab,
          a0, a1, lb0, lb1, a_tl, ix0, ix1, g0, g1, o0, o1,
          sa0, sa1, sl0, sl1, si0, si1, sg0, sg1, so0, so1, bsem):
    dw = D // 2
    wid = lax.axis_index("s") * NC + lax.axis_index("c")

    iotas = [lax.iota(jnp.int32, 16) + 16 * g for g in range(BLK_A // 16)]
    cols = [jnp.full((16,), k, jnp.int32) for k in range(dw)]

    a_bufs = (a0, a1)
    lb_bufs = (lb0, lb1)
    sa = (sa0, sa1)
    sl = (sl0, sl1)

    nblk = V // BLK_A
    tail = V - nblk * BLK_A
    per = nblk // NW
    na = per + 1 + (per + 1) % 2

    def blk_of(i):
        return (wid * per + i) % nblk

    def a_in(i, u):
        return pltpu.make_async_copy(
            wT.at[:, pl.ds(blk_of(i) * BLK_A, BLK_A)], a_bufs[u], sa[u])

    def a_out(i, u):
        return pltpu.make_async_copy(
            lb_bufs[u], ltab.at[pl.ds(blk_of(i) * BLK_A, BLK_A)], sl[u])

    def transpose_a(u):
        src = a_bufs[u].bitcast(jnp.int32)
        dst = lb_bufs[u]
        for k in range(dw):
            for g in range(BLK_A // 16):
                plsc.store_scatter(dst, [iotas[g], cols[k]],
                                   src[k, pl.ds(16 * g, 16)])

    a_in(0, 0).start()

    def phase_a(j, _):
        for u in (0, 1):
            i = 2 * j + u

            @pl.when(i + 1 < na)
            def _():
                a_in(i + 1, 1 - u).start()

            a_in(i, u).wait()

            @pl.when(i >= 2)
            def _():
                a_out(i - 2, u).wait()

            transpose_a(u)
            a_out(i, u).start()
        return 0

    lax.fori_loop(0, na // 2, phase_a, 0)
    a_out(na - 2, 0).wait()
    a_out(na - 1, 1).wait()

    @pl.when(wid == NW - 1)
    def _tail():
        pltpu.async_copy(wtail, a_tl, sa0).wait()
        t32 = a_tl.bitcast(jnp.int32)
        for k in range(dw):
            for g in range(tail // 16):
                plsc.store_scatter(lb0, [iotas[g], cols[k]],
                                   t32[k, pl.ds(16 * g, 16)])
        pltpu.async_copy(lb0.at[pl.ds(0, tail)],
                         ltab.at[pl.ds(nblk * BLK_A, tail)], sl0).wait()

    for tc in range(NC):
        for ts in range(NS):
            pl.semaphore_signal(bsem, 1, device_id={"c": tc, "s": ts})
    pl.semaphore_wait(bsem, NW)

    ntb = B // BLK_B
    nb = (H * ntb) // NW
    ix = (ix0, ix1)
    gb = (g0, g1)
    ob = (o0, o1)
    si = (si0, si1)
    sg = (sg0, sg1)
    so = (so0, so1)

    def b_idx_in(i, u):
        fb = wid * nb + i
        return pltpu.make_async_copy(
            idxT.at[fb // ntb, pl.ds((fb % ntb) * BLK_B, BLK_B)], ix[u], si[u])

    def b_gather(u):
        return pltpu.make_async_copy(ltab.at[ix[u]], gb[u], sg[u])

    def b_out(i, u):
        fb = wid * nb + i
        return pltpu.make_async_copy(
            ob[u], out.at[fb // ntb, :, pl.ds((fb % ntb) * BLK_B, BLK_B)],
            so[u])

    def transpose_b(u):
        src = gb[u]
        dst = ob[u].bitcast(jnp.int32)
        for k in range(dw):
            for g in range(BLK_B // 16):
                dst[k, pl.ds(16 * g, 16)] = plsc.load_gather(
                    src, [iotas[g], cols[k]])

    b_idx_in(0, 0).start()
    b_idx_in(0, 0).wait()
    b_gather(0).start()
    b_idx_in(1, 1).start()

    def phase_b(j, _):
        for u in (0, 1):
            i = 2 * j + u

            @pl.when(i + 1 < nb)
            def _():
                b_idx_in(i + 1, 1 - u).wait()
                b_gather(1 - u).start()

            b_gather(u).wait()

            @pl.when(i + 2 < nb)
            def _():
                b_idx_in(i + 2, u).start()

            @pl.when(i >= 2)
            def _():
                b_out(i - 2, u).wait()

            transpose_b(u)
            b_out(i, u).start()
        return 0

    lax.fori_loop(0, nb // 2, phase_b, 0)
    b_out(nb - 2, 0).wait()
    b_out(nb - 1, 1).wait()


def kernel(indices, weight):
    B, H = indices.shape
    V, D = weight.shape
    dw = D // 2
    assert B % LANE == 0 and (H * (B // LANE)) % NW == 0

    nblk = V // BLK_A
    tail = V - nblk * BLK_A
    wT = weight.T
    wtail = lax.slice(wT, (0, nblk * BLK_A), (D, V))

    mesh = plsc.VectorSubcoreMesh(core_axis_name="c", subcore_axis_name="s")
    body = functools.partial(_body, V, D, B, H)
    out = pl.kernel(
        body,
        out_type=jax.ShapeDtypeStruct((H, D, B), weight.dtype),
        mesh=mesh,
        scratch_types=[
            pltpu.HBM((V, dw), jnp.int32),
            pltpu.VMEM((D, BLK_A), weight.dtype),
            pltpu.VMEM((D, BLK_A), weight.dtype),
            pltpu.VMEM((BLK_A, dw), jnp.int32),
            pltpu.VMEM((BLK_A, dw), jnp.int32),
            pltpu.VMEM((D, tail), weight.dtype),
            pltpu.VMEM((BLK_B,), jnp.int32),
            pltpu.VMEM((BLK_B,), jnp.int32),
            pltpu.VMEM((BLK_B, dw), jnp.int32),
            pltpu.VMEM((BLK_B, dw), jnp.int32),
            pltpu.VMEM((D, BLK_B), weight.dtype),
            pltpu.VMEM((D, BLK_B), weight.dtype),
            pltpu.SemaphoreType.DMA,
            pltpu.SemaphoreType.DMA,
            pltpu.SemaphoreType.DMA,
            pltpu.SemaphoreType.DMA,
            pltpu.SemaphoreType.DMA,
            pltpu.SemaphoreType.DMA,
            pltpu.SemaphoreType.DMA,
            pltpu.SemaphoreType.DMA,
            pltpu.SemaphoreType.DMA,
            pltpu.SemaphoreType.DMA,
            pltpu.SemaphoreType.REGULAR,
        ],
        compiler_params=pltpu.CompilerParams(use_tc_tiling_on_sc=True,
                                             needs_layout_passes=False),
    )(wT, indices.T, wtail)
    return out.transpose(2, 0, 1)
